# Initial kernel scaffold; baseline (speedup 1.0000x reference)
import jax
import jax.numpy as jnp
from jax.experimental import pallas as pl


def kernel(descriptor, r_array_q, theta_array_q, sum_points):
    raise NotImplementedError("write your pallas kernel here")


# speedup vs baseline: 40.7251x; 40.7251x over previous
import functools

import jax
import jax.numpy as jnp
from jax import lax
from jax.experimental import pallas as pl
from jax.experimental.pallas import tpu as pltpu
from jax.experimental.pallas import tpu_sc as plsc

_N_THETA = 16
_N_BINS = 128
_LANES = 16


def kernel(descriptor, r_array_q, theta_array_q, sum_points):
    B, N, NB = descriptor.shape
    R = B * N
    NW = 32
    G = _LANES
    rows_per_w = R // NW
    groups_per_w = rows_per_w // G
    chunks_per_group = (G * N) // _LANES

    r_flat = r_array_q.reshape(R * N)
    t_flat = theta_array_q.reshape(R * N)
    d_flat = descriptor.reshape(R * NB)
    inv = 1.0 / sum_points.astype(jnp.float32)
    inv_w = jnp.repeat(inv, NW // B)
    inv_splat = jnp.broadcast_to(inv_w[:, None], (NW, _LANES))

    mesh = plsc.VectorSubcoreMesh(core_axis_name="c", subcore_axis_name="s")

    @functools.partial(
        pl.kernel,
        out_type=jax.ShapeDtypeStruct((R * NB,), jnp.float32),
        mesh=mesh,
        scratch_types=[
            pltpu.VMEM((G * N,), jnp.int32),
            pltpu.VMEM((G * N,), jnp.int32),
            pltpu.VMEM((G * NB,), jnp.float32),
            pltpu.VMEM((_LANES,), jnp.float32),
        ],
        compiler_params=pltpu.CompilerParams(needs_layout_passes=False),
    )
    def run(d_hbm, r_hbm, t_hbm, inv_hbm, out_hbm, rbuf, tbuf, acc, invv):
        wid = lax.axis_index("s") * 2 + lax.axis_index("c")
        pltpu.sync_copy(inv_hbm.at[wid], invv)
        ival = invv[...]

        def do_group(g, _):
            row_base = (wid * groups_per_w + g) * G
            pltpu.sync_copy(r_hbm.at[pl.ds(row_base * N, G * N)], rbuf)
            pltpu.sync_copy(t_hbm.at[pl.ds(row_base * N, G * N)], tbuf)
            pltpu.sync_copy(d_hbm.at[pl.ds(row_base * NB, G * NB)], acc)

            def do_chunk(i, _):
                off = pl.multiple_of(i * _LANES, _LANES)
                rv = rbuf[pl.ds(off, _LANES)]
                tv = tbuf[pl.ds(off, _LANES)]
                bins = rv * _N_THETA + tv
                row_off = (i // (N // _LANES)) * _N_BINS
                sidx = bins + jnp.full((_LANES,), row_off, jnp.int32)
                plsc.addupdate_scatter(acc, [sidx], ival)
                return 0

            lax.fori_loop(0, chunks_per_group, do_chunk, 0)
            pltpu.sync_copy(acc, out_hbm.at[pl.ds(row_base * NB, G * NB)])
            return 0

        lax.fori_loop(0, groups_per_w, do_group, 0)

    out = run(d_flat, r_flat, t_flat, inv_splat)
    return out.reshape(B, N, NB)


# speedup vs baseline: 47.2219x; 1.1595x over previous
import functools

import jax
import jax.numpy as jnp
from jax import lax
from jax.experimental import pallas as pl
from jax.experimental.pallas import tpu as pltpu
from jax.experimental.pallas import tpu_sc as plsc

_N_THETA = 16
_N_BINS = 128
_LANES = 16


def kernel(descriptor, r_array_q, theta_array_q, sum_points):
    B, N, NB = descriptor.shape
    R = B * N
    NW = 32
    G = _LANES
    rows_per_w = R // NW
    groups_per_w = rows_per_w // G
    n_iters = groups_per_w // 2

    r_flat = r_array_q.reshape(R * N)
    t_flat = theta_array_q.reshape(R * N)
    d_flat = descriptor.reshape(R * NB)
    inv = 1.0 / sum_points.astype(jnp.float32)
    inv_w = jnp.repeat(inv, NW // B)
    inv_splat = jnp.broadcast_to(inv_w[:, None], (NW, _LANES))

    mesh = plsc.VectorSubcoreMesh(core_axis_name="c", subcore_axis_name="s")

    @functools.partial(
        pl.kernel,
        out_type=jax.ShapeDtypeStruct((R * NB,), jnp.float32),
        mesh=mesh,
        scratch_types=[
            pltpu.VMEM((G * N,), jnp.int32),
            pltpu.VMEM((G * N,), jnp.int32),
            pltpu.VMEM((G * N,), jnp.int32),
            pltpu.VMEM((G * N,), jnp.int32),
            pltpu.VMEM((G * NB,), jnp.float32),
            pltpu.VMEM((G * NB,), jnp.float32),
            pltpu.VMEM((_LANES,), jnp.float32),
            pltpu.SemaphoreType.DMA,
            pltpu.SemaphoreType.DMA,
            pltpu.SemaphoreType.DMA,
            pltpu.SemaphoreType.DMA,
            pltpu.SemaphoreType.DMA,
            pltpu.SemaphoreType.DMA,
        ],
        compiler_params=pltpu.CompilerParams(needs_layout_passes=False),
    )
    def run(d_hbm, r_hbm, t_hbm, inv_hbm, out_hbm,
            rb0, rb1, tb0, tb1, acc0, acc1, invv,
            isem0, isem1, dsem0, dsem1, osem0, osem1):
        wid = lax.axis_index("s") * 2 + lax.axis_index("c")
        pltpu.sync_copy(inv_hbm.at[wid], invv)
        ival = invv[...]
        rb = (rb0, rb1)
        tb = (tb0, tb1)
        acc = (acc0, acc1)
        isem = (isem0, isem1)
        dsem = (dsem0, dsem1)
        osem = (osem0, osem1)
        w_row0 = wid * rows_per_w

        def fire_in(g, buf):
            base = (w_row0 + g * G) * N
            pltpu.async_copy(r_hbm.at[pl.ds(base, G * N)], rb[buf], isem[buf])
            pltpu.async_copy(t_hbm.at[pl.ds(base, G * N)], tb[buf], isem[buf])

        def fire_desc(g, buf):
            base = (w_row0 + g * G) * NB
            pltpu.async_copy(d_hbm.at[pl.ds(base, G * NB)], acc[buf], dsem[buf])

        def wait_in(g, buf):
            base = (w_row0 + g * G) * N
            pltpu.make_async_copy(r_hbm.at[pl.ds(base, G * N)], rb[buf], isem[buf]).wait()
            pltpu.make_async_copy(t_hbm.at[pl.ds(base, G * N)], tb[buf], isem[buf]).wait()
            base_d = (w_row0 + g * G) * NB
            pltpu.make_async_copy(d_hbm.at[pl.ds(base_d, G * NB)], acc[buf], dsem[buf]).wait()

        def fire_out(g, buf):
            base = (w_row0 + g * G) * NB
            pltpu.async_copy(acc[buf], out_hbm.at[pl.ds(base, G * NB)], osem[buf])

        def wait_out(buf):
            pltpu.make_async_copy(d_hbm.at[pl.ds(0, G * NB)], acc[buf], osem[buf]).wait()

        def compute(buf):
            rbr, tbr, accr = rb[buf], tb[buf], acc[buf]

            def row_body(r, carry):
                base = r * N
                roff = jnp.full((_LANES,), r * NB, jnp.int32)
                for c in range(N // _LANES):
                    off = pl.multiple_of(base + c * _LANES, _LANES)
                    rv = rbr[pl.ds(off, _LANES)]
                    tv = tbr[pl.ds(off, _LANES)]
                    sidx = rv * _N_THETA + tv + roff
                    plsc.addupdate_scatter(accr, [sidx], ival)
                return carry

            lax.fori_loop(0, G, row_body, 0)

        fire_in(0, 0)
        fire_desc(0, 0)

        def step(k, carry):
            g0 = 2 * k
            g1 = g0 + 1
            fire_in(g1, 1)
            wait_in(g0, 0)
            compute(0)

            @pl.when(k >= 1)
            def _():
                wait_out(1)
            fire_desc(g1, 1)
            fire_out(g0, 0)

            @pl.when(k < n_iters - 1)
            def _():
                fire_in(g0 + 2, 0)
            wait_in(g1, 1)
            compute(1)

            @pl.when(k < n_iters - 1)
            def _():
                wait_out(0)
                fire_desc(g0 + 2, 0)
            fire_out(g1, 1)
            return carry

        lax.fori_loop(0, n_iters, step, 0)
        wait_out(0)
        wait_out(1)

    out = run(d_flat, r_flat, t_flat, inv_splat)
    return out.reshape(B, N, NB)


# speedup vs baseline: 92.8934x; 1.9672x over previous
import functools

import jax
import jax.numpy as jnp
from jax import lax
from jax.experimental import pallas as pl
from jax.experimental.pallas import tpu as pltpu
from jax.experimental.pallas import tpu_sc as plsc

_N_THETA = 16
_N_BINS = 128
_LANES = 16


def kernel(descriptor, r_array_q, theta_array_q, sum_points):
    B, N, NB = descriptor.shape
    R = B * N
    NW = 32
    G = _LANES
    rows_per_w = R // NW
    groups_per_w = rows_per_w // G
    n_iters = groups_per_w // 2
    w_per_b = N // rows_per_w

    inv = 1.0 / sum_points.astype(jnp.float32)
    inv_w = jnp.repeat(inv, NW // B)
    inv_splat = jnp.broadcast_to(inv_w[:, None], (NW, _LANES))

    mesh = plsc.VectorSubcoreMesh(core_axis_name="c", subcore_axis_name="s")

    @functools.partial(
        pl.kernel,
        out_type=jax.ShapeDtypeStruct((B, N, NB), jnp.float32),
        mesh=mesh,
        scratch_types=[
            pltpu.VMEM((G, N), jnp.int32),
            pltpu.VMEM((G, N), jnp.int32),
            pltpu.VMEM((G, N), jnp.int32),
            pltpu.VMEM((G, N), jnp.int32),
            pltpu.VMEM((G, NB), jnp.float32),
            pltpu.VMEM((G, NB), jnp.float32),
            pltpu.VMEM((_LANES,), jnp.float32),
            pltpu.SemaphoreType.DMA,
            pltpu.SemaphoreType.DMA,
            pltpu.SemaphoreType.DMA,
            pltpu.SemaphoreType.DMA,
            pltpu.SemaphoreType.DMA,
            pltpu.SemaphoreType.DMA,
        ],
        compiler_params=pltpu.CompilerParams(needs_layout_passes=False),
    )
    def run(d_hbm, r_hbm, t_hbm, inv_hbm, out_hbm,
            rb0, rb1, tb0, tb1, acc0, acc1, invv,
            isem0, isem1, dsem0, dsem1, osem0, osem1):
        wid = lax.axis_index("s") * 2 + lax.axis_index("c")
        pltpu.sync_copy(inv_hbm.at[wid], invv)
        ival = invv[...]
        rb = (rb0, rb1)
        tb = (tb0, tb1)
        acc = (acc0, acc1)
        isem = (isem0, isem1)
        dsem = (dsem0, dsem1)
        osem = (osem0, osem1)
        bno = wid // w_per_b
        w_lr0 = (wid % w_per_b) * rows_per_w

        def fire_in(g, buf):
            lr = w_lr0 + g * G
            pltpu.async_copy(r_hbm.at[bno, pl.ds(lr, G)], rb[buf], isem[buf])
            pltpu.async_copy(t_hbm.at[bno, pl.ds(lr, G)], tb[buf], isem[buf])

        def fire_desc(g, buf):
            lr = w_lr0 + g * G
            pltpu.async_copy(d_hbm.at[bno, pl.ds(lr, G)], acc[buf], dsem[buf])

        def wait_in(g, buf):
            lr = w_lr0 + g * G
            pltpu.make_async_copy(r_hbm.at[bno, pl.ds(lr, G)], rb[buf], isem[buf]).wait()
            pltpu.make_async_copy(t_hbm.at[bno, pl.ds(lr, G)], tb[buf], isem[buf]).wait()
            pltpu.make_async_copy(d_hbm.at[bno, pl.ds(lr, G)], acc[buf], dsem[buf]).wait()

        def fire_out(g, buf):
            lr = w_lr0 + g * G
            pltpu.async_copy(acc[buf], out_hbm.at[bno, pl.ds(lr, G)], osem[buf])

        def wait_out(buf):
            pltpu.make_async_copy(d_hbm.at[0, pl.ds(0, G)], acc[buf], osem[buf]).wait()

        def compute(buf):
            rbr, tbr, accr = rb[buf], tb[buf], acc[buf]

            @plsc.parallel_loop(0, G, 1)
            def row_body(r):
                rsplat = jnp.full((_LANES,), r, jnp.int32)
                for c in range(N // _LANES):
                    rv = rbr[r, pl.ds(c * _LANES, _LANES)]
                    tv = tbr[r, pl.ds(c * _LANES, _LANES)]
                    bins = rv * _N_THETA + tv
                    plsc.addupdate_scatter(accr, [rsplat, bins], ival)

        fire_in(0, 0)
        fire_desc(0, 0)

        def step(k, carry):
            g0 = 2 * k
            g1 = g0 + 1
            fire_in(g1, 1)
            wait_in(g0, 0)
            compute(0)

            @pl.when(k >= 1)
            def _():
                wait_out(1)
            fire_desc(g1, 1)
            fire_out(g0, 0)

            @pl.when(k < n_iters - 1)
            def _():
                fire_in(g0 + 2, 0)
            wait_in(g1, 1)
            compute(1)

            @pl.when(k < n_iters - 1)
            def _():
                wait_out(0)
                fire_desc(g0 + 2, 0)
            fire_out(g1, 1)
            return carry

        lax.fori_loop(0, n_iters, step, 0)
        wait_out(0)
        wait_out(1)

    return run(descriptor, r_array_q, theta_array_q, inv_splat)


# speedup vs baseline: 101.3187x; 1.0907x over previous
import functools

import jax
import jax.numpy as jnp
from jax import lax
from jax.experimental import pallas as pl
from jax.experimental.pallas import tpu as pltpu
from jax.experimental.pallas import tpu_sc as plsc

_N_THETA = 16
_N_BINS = 128
_LANES = 16


def kernel(descriptor, r_array_q, theta_array_q, sum_points):
    B, N, NB = descriptor.shape
    R = B * N
    NW = 32
    G = _LANES
    rows_per_w = R // NW
    groups_per_w = rows_per_w // G
    n_iters = groups_per_w // 2
    w_per_b = N // rows_per_w

    r2 = r_array_q.reshape(R, N)
    t2 = theta_array_q.reshape(R, N)
    d2 = descriptor.reshape(R, NB)

    inv = 1.0 / sum_points.astype(jnp.float32)
    inv_w = jnp.repeat(inv, NW // B)
    inv_splat = jnp.broadcast_to(inv_w[:, None], (NW, _LANES))

    mesh = plsc.VectorSubcoreMesh(core_axis_name="c", subcore_axis_name="s")

    @functools.partial(
        pl.kernel,
        out_type=jax.ShapeDtypeStruct((R, NB), jnp.float32),
        mesh=mesh,
        scratch_types=[
            pltpu.VMEM((G, N), jnp.int32),
            pltpu.VMEM((G, N), jnp.int32),
            pltpu.VMEM((G, N), jnp.int32),
            pltpu.VMEM((G, N), jnp.int32),
            pltpu.VMEM((NB, G), jnp.float32),
            pltpu.VMEM((G, NB), jnp.float32),
            pltpu.VMEM((G, NB), jnp.float32),
            pltpu.VMEM((_LANES,), jnp.float32),
            pltpu.SemaphoreType.DMA,
            pltpu.SemaphoreType.DMA,
            pltpu.SemaphoreType.DMA,
            pltpu.SemaphoreType.DMA,
            pltpu.SemaphoreType.DMA,
            pltpu.SemaphoreType.DMA,
        ],
        compiler_params=pltpu.CompilerParams(needs_layout_passes=False),
    )
    def run(d_hbm, r_hbm, t_hbm, inv_hbm, out_hbm,
            rb0, rb1, tb0, tb1, acct, ob0, ob1, invv,
            isem0, isem1, dsem0, dsem1, osem0, osem1):
        wid = lax.axis_index("s") * 2 + lax.axis_index("c")
        pltpu.sync_copy(inv_hbm.at[wid], invv)
        ival = invv[...]
        rb = (rb0, rb1)
        tb = (tb0, tb1)
        ob = (ob0, ob1)
        isem = (isem0, isem1)
        dsem = (dsem0, dsem1)
        osem = (osem0, osem1)
        w_row0 = wid * rows_per_w
        iota = lax.iota(jnp.int32, _LANES)
        zero16 = jnp.zeros((_LANES,), jnp.float32)

        def fire_in(g, buf):
            row = w_row0 + g * G
            pltpu.async_copy(r_hbm.at[pl.ds(row, G)], rb[buf], isem[buf])
            pltpu.async_copy(t_hbm.at[pl.ds(row, G)], tb[buf], isem[buf])

        def fire_desc(g, buf):
            row = w_row0 + g * G
            pltpu.async_copy(d_hbm.at[pl.ds(row, G)], ob[buf], dsem[buf])

        def wait_in(g, buf):
            row = w_row0 + g * G
            pltpu.make_async_copy(r_hbm.at[pl.ds(row, G)], rb[buf], isem[buf]).wait()
            pltpu.make_async_copy(t_hbm.at[pl.ds(row, G)], tb[buf], isem[buf]).wait()
            pltpu.make_async_copy(d_hbm.at[pl.ds(row, G)], ob[buf], dsem[buf]).wait()

        def fire_out(g, buf):
            row = w_row0 + g * G
            pltpu.async_copy(ob[buf], out_hbm.at[pl.ds(row, G)], osem[buf])

        def wait_out(buf):
            pltpu.make_async_copy(d_hbm.at[pl.ds(0, G)], ob[buf], osem[buf]).wait()

        def compute(buf):
            rbr, tbr, obr = rb[buf], tb[buf], ob[buf]

            @plsc.parallel_loop(0, NB, 1)
            def zero_body(bin_):
                acct[bin_, :] = zero16

            @plsc.parallel_loop(0, N, 1, carry=iota)
            def col_body(j, acol):
                rv = plsc.load_gather(rbr, [iota, acol])
                tv = plsc.load_gather(tbr, [iota, acol])
                bins = (rv << 4) + tv
                plsc.addupdate_scatter(acct, [bins, iota], ival)
                return (acol + 1) & (N - 1)

            @plsc.parallel_loop(0, NB, 1)
            def trans_body(s):
                t16 = s & (NB - G)
                d = s & (G - 1)
                bvec = ((iota + d) & (G - 1)) + t16
                v = plsc.load_gather(acct, [bvec, iota])
                plsc.addupdate_scatter(obr, [iota, bvec], v)

        fire_in(0, 0)
        fire_desc(0, 0)

        def step(k, carry):
            g0 = 2 * k
            g1 = g0 + 1
            fire_in(g1, 1)
            wait_in(g0, 0)
            compute(0)

            @pl.when(k >= 1)
            def _():
                wait_out(1)
            fire_desc(g1, 1)
            fire_out(g0, 0)

            @pl.when(k < n_iters - 1)
            def _():
                fire_in(g0 + 2, 0)
            wait_in(g1, 1)
            compute(1)

            @pl.when(k < n_iters - 1)
            def _():
                wait_out(0)
                fire_desc(g0 + 2, 0)
            fire_out(g1, 1)
            return carry

        lax.fori_loop(0, n_iters, step, 0)
        wait_out(0)
        wait_out(1)

    return run(d2, r2, t2, inv_splat).reshape(B, N, NB)


# speedup vs baseline: 128.8499x; 1.2717x over previous
import functools

import jax
import jax.numpy as jnp
from jax import lax
from jax.experimental import pallas as pl
from jax.experimental.pallas import tpu as pltpu
from jax.experimental.pallas import tpu_sc as plsc

_N_THETA = 16
_N_BINS = 128
_LANES = 16


def kernel(descriptor, r_array_q, theta_array_q, sum_points):
    B, N, NB = descriptor.shape
    R = B * N
    NW = 32
    G = _LANES
    rows_per_w = R // NW
    groups_per_w = rows_per_w // G
    n_iters = groups_per_w // 2
    w_per_b = N // rows_per_w

    r2 = r_array_q.reshape(R, N)
    t2 = theta_array_q.reshape(R, N)
    d2 = descriptor.reshape(R, NB)

    inv = 1.0 / sum_points.astype(jnp.float32)
    inv_w = jnp.repeat(inv, NW // B)
    inv_splat = jnp.broadcast_to(inv_w[:, None], (NW, _LANES))

    mesh = plsc.VectorSubcoreMesh(core_axis_name="c", subcore_axis_name="s")

    @functools.partial(
        pl.kernel,
        out_type=jax.ShapeDtypeStruct((R, NB), jnp.float32),
        mesh=mesh,
        scratch_types=[
            pltpu.VMEM((G, N), jnp.int32),
            pltpu.VMEM((G, N), jnp.int32),
            pltpu.VMEM((G, N), jnp.int32),
            pltpu.VMEM((G, N), jnp.int32),
            pltpu.VMEM((NB, G), jnp.float32),
            pltpu.VMEM((G, NB), jnp.float32),
            pltpu.VMEM((G, NB), jnp.float32),
            pltpu.VMEM((_LANES,), jnp.float32),
            pltpu.SemaphoreType.DMA,
            pltpu.SemaphoreType.DMA,
            pltpu.SemaphoreType.DMA,
            pltpu.SemaphoreType.DMA,
            pltpu.SemaphoreType.DMA,
            pltpu.SemaphoreType.DMA,
        ],
        compiler_params=pltpu.CompilerParams(needs_layout_passes=False),
    )
    def run(d_hbm, r_hbm, t_hbm, inv_hbm, out_hbm,
            rb0, rb1, tb0, tb1, acct, ob0, ob1, invv,
            isem0, isem1, dsem0, dsem1, osem0, osem1):
        wid = lax.axis_index("s") * 2 + lax.axis_index("c")
        pltpu.sync_copy(inv_hbm.at[wid], invv)
        ival = invv[...]
        rb = (rb0, rb1)
        tb = (tb0, tb1)
        ob = (ob0, ob1)
        isem = (isem0, isem1)
        dsem = (dsem0, dsem1)
        osem = (osem0, osem1)
        w_row0 = wid * rows_per_w
        iota = lax.iota(jnp.int32, _LANES)
        zero16 = jnp.zeros((_LANES,), jnp.float32)

        def fire_in(g, buf):
            row = w_row0 + g * G
            pltpu.async_copy(r_hbm.at[pl.ds(row, G)], rb[buf], isem[buf])
            pltpu.async_copy(t_hbm.at[pl.ds(row, G)], tb[buf], isem[buf])

        def fire_desc(g, buf):
            row = w_row0 + g * G
            pltpu.async_copy(d_hbm.at[pl.ds(row, G)], ob[buf], dsem[buf])

        def wait_in(g, buf):
            row = w_row0 + g * G
            pltpu.make_async_copy(r_hbm.at[pl.ds(row, G)], rb[buf], isem[buf]).wait()
            pltpu.make_async_copy(t_hbm.at[pl.ds(row, G)], tb[buf], isem[buf]).wait()
            pltpu.make_async_copy(d_hbm.at[pl.ds(row, G)], ob[buf], dsem[buf]).wait()

        def fire_out(g, buf):
            row = w_row0 + g * G
            pltpu.async_copy(ob[buf], out_hbm.at[pl.ds(row, G)], osem[buf])

        def wait_out(buf):
            pltpu.make_async_copy(d_hbm.at[pl.ds(0, G)], ob[buf], osem[buf]).wait()

        def compute(buf):
            rbr, tbr, obr = rb[buf], tb[buf], ob[buf]

            @plsc.parallel_loop(0, NB, 1)
            def zero_body(bin_):
                acct[bin_, :] = zero16

            @plsc.parallel_loop(0, N, 2)
            def col_body(j):
                jvec = jnp.full((_LANES,), j, jnp.int32) + iota
                for u in range(2):
                    acol = (jvec + u) & (N - 1)
                    rv = plsc.load_gather(rbr, [iota, acol])
                    tv = plsc.load_gather(tbr, [iota, acol])
                    bins = (rv << 4) + tv
                    plsc.addupdate_scatter(acct, [bins, iota], ival)

            @plsc.parallel_loop(0, NB, 1)
            def trans_body(s):
                t16 = s & (NB - G)
                d = s & (G - 1)
                bvec = ((iota + d) & (G - 1)) + t16
                v = plsc.load_gather(acct, [bvec, iota])
                plsc.addupdate_scatter(obr, [iota, bvec], v)

        fire_in(0, 0)
        fire_desc(0, 0)

        def step(k, carry):
            g0 = 2 * k
            g1 = g0 + 1
            fire_in(g1, 1)
            wait_in(g0, 0)
            compute(0)

            @pl.when(k >= 1)
            def _():
                wait_out(1)
            fire_desc(g1, 1)
            fire_out(g0, 0)

            @pl.when(k < n_iters - 1)
            def _():
                fire_in(g0 + 2, 0)
            wait_in(g1, 1)
            compute(1)

            @pl.when(k < n_iters - 1)
            def _():
                wait_out(0)
                fire_desc(g0 + 2, 0)
            fire_out(g1, 1)
            return carry

        lax.fori_loop(0, n_iters, step, 0)
        wait_out(0)
        wait_out(1)

    return run(d2, r2, t2, inv_splat).reshape(B, N, NB)


# speedup vs baseline: 129.0189x; 1.0013x over previous
import functools

import jax
import jax.numpy as jnp
from jax import lax
from jax.experimental import pallas as pl
from jax.experimental.pallas import tpu as pltpu
from jax.experimental.pallas import tpu_sc as plsc

_N_THETA = 16
_N_BINS = 128
_LANES = 16


def kernel(descriptor, r_array_q, theta_array_q, sum_points):
    B, N, NB = descriptor.shape
    R = B * N
    NW = 32
    G = _LANES
    rows_per_w = R // NW
    groups_per_w = rows_per_w // G
    n_iters = groups_per_w // 2
    w_per_b = N // rows_per_w

    r2 = r_array_q.reshape(R, N)
    t2 = theta_array_q.reshape(R, N)
    d2 = descriptor.reshape(R, NB)

    inv = 1.0 / sum_points.astype(jnp.float32)
    inv_w = jnp.repeat(inv, NW // B)
    inv_splat = jnp.broadcast_to(inv_w[:, None], (NW, _LANES))

    mesh = plsc.VectorSubcoreMesh(core_axis_name="c", subcore_axis_name="s")

    @functools.partial(
        pl.kernel,
        out_type=jax.ShapeDtypeStruct((R, NB), jnp.float32),
        mesh=mesh,
        scratch_types=[
            pltpu.VMEM((G, N), jnp.int32),
            pltpu.VMEM((G, N), jnp.int32),
            pltpu.VMEM((G, N), jnp.int32),
            pltpu.VMEM((G, N), jnp.int32),
            pltpu.VMEM((NB, G), jnp.float32),
            pltpu.VMEM((G, NB), jnp.float32),
            pltpu.VMEM((G, NB), jnp.float32),
            pltpu.VMEM((_LANES,), jnp.float32),
            pltpu.SemaphoreType.DMA,
            pltpu.SemaphoreType.DMA,
            pltpu.SemaphoreType.DMA,
            pltpu.SemaphoreType.DMA,
            pltpu.SemaphoreType.DMA,
            pltpu.SemaphoreType.DMA,
        ],
        compiler_params=pltpu.CompilerParams(needs_layout_passes=False),
    )
    def run(d_hbm, r_hbm, t_hbm, inv_hbm, out_hbm,
            rb0, rb1, tb0, tb1, acct, ob0, ob1, invv,
            isem0, isem1, dsem0, dsem1, osem0, osem1):
        wid = lax.axis_index("s") * 2 + lax.axis_index("c")
        pltpu.sync_copy(inv_hbm.at[wid], invv)
        ival = invv[...]
        rb = (rb0, rb1)
        tb = (tb0, tb1)
        ob = (ob0, ob1)
        isem = (isem0, isem1)
        dsem = (dsem0, dsem1)
        osem = (osem0, osem1)
        w_row0 = wid * rows_per_w
        iota = lax.iota(jnp.int32, _LANES)
        zero16 = jnp.zeros((_LANES,), jnp.float32)

        def fire_in(g, buf):
            row = w_row0 + g * G
            pltpu.async_copy(r_hbm.at[pl.ds(row, G)], rb[buf], isem[buf])
            pltpu.async_copy(t_hbm.at[pl.ds(row, G)], tb[buf], isem[buf])

        def fire_desc(g, buf):
            row = w_row0 + g * G
            pltpu.async_copy(d_hbm.at[pl.ds(row, G)], ob[buf], dsem[buf])

        def wait_in(g, buf):
            row = w_row0 + g * G
            pltpu.make_async_copy(r_hbm.at[pl.ds(row, G)], rb[buf], isem[buf]).wait()
            pltpu.make_async_copy(t_hbm.at[pl.ds(row, G)], tb[buf], isem[buf]).wait()
            pltpu.make_async_copy(d_hbm.at[pl.ds(row, G)], ob[buf], dsem[buf]).wait()

        def fire_out(g, buf):
            row = w_row0 + g * G
            pltpu.async_copy(ob[buf], out_hbm.at[pl.ds(row, G)], osem[buf])

        def wait_out(buf):
            pltpu.make_async_copy(d_hbm.at[pl.ds(0, G)], ob[buf], osem[buf]).wait()

        def compute(buf):
            rbr, tbr, obr = rb[buf], tb[buf], ob[buf]

            @plsc.parallel_loop(0, NB, 1)
            def zero_body(bin_):
                acct[bin_, :] = zero16

            @plsc.parallel_loop(0, N, 8)
            def col_body(j):
                jvec = jnp.full((_LANES,), j, jnp.int32) + iota
                for u in range(8):
                    acol = (jvec + u) & (N - 1)
                    rv = plsc.load_gather(rbr, [iota, acol])
                    tv = plsc.load_gather(tbr, [iota, acol])
                    bins = (rv << 4) + tv
                    plsc.addupdate_scatter(acct, [bins, iota], ival)

            @plsc.parallel_loop(0, NB, 1)
            def trans_body(s):
                t16 = s & (NB - G)
                d = s & (G - 1)
                bvec = ((iota + d) & (G - 1)) + t16
                v = plsc.load_gather(acct, [bvec, iota])
                plsc.addupdate_scatter(obr, [iota, bvec], v)

        fire_in(0, 0)
        fire_desc(0, 0)

        def step(k, carry):
            g0 = 2 * k
            g1 = g0 + 1
            fire_in(g1, 1)
            wait_in(g0, 0)
            compute(0)

            @pl.when(k >= 1)
            def _():
                wait_out(1)
            fire_desc(g1, 1)
            fire_out(g0, 0)

            @pl.when(k < n_iters - 1)
            def _():
                fire_in(g0 + 2, 0)
            wait_in(g1, 1)
            compute(1)

            @pl.when(k < n_iters - 1)
            def _():
                wait_out(0)
                fire_desc(g0 + 2, 0)
            fire_out(g1, 1)
            return carry

        lax.fori_loop(0, n_iters, step, 0)
        wait_out(0)
        wait_out(1)

    return run(d2, r2, t2, inv_splat).reshape(B, N, NB)
